# Initial kernel scaffold; baseline (speedup 1.0000x reference)
import jax
import jax.numpy as jnp
from jax.experimental import pallas as pl


def kernel(x, edge_index, W1a, b1a, W1b, b1b, g1, be1, W2a, b2a, W2b, b2b, g2, be2):
    raise NotImplementedError("write your pallas kernel here")


# speedup vs baseline: 5.5638x; 5.5638x over previous
import functools

import jax
import jax.numpy as jnp
from jax import lax
from jax.experimental import pallas as pl
from jax.experimental.pallas import tpu as pltpu
from jax.experimental.pallas import tpu_sc as plsc

N_NODES = 10000
N_PAD = 10112
D_IN = 128
DIM = 32
BN_EPS = 1e-5
N_EDGES = 320000

NC = 2
NS = 16
NW = NC * NS
E_PER_W = N_EDGES // NW
CHUNK = 128
NCHUNK = 80
E_PAD_W = NCHUNK * CHUNK
NBUF = 2
ROWS_PER_TILE = N_PAD // NS

_mesh = plsc.VectorSubcoreMesh(core_axis_name="c", subcore_axis_name="s")


def _make_segsum(d):
    @functools.partial(
        pl.kernel,
        mesh=_mesh,
        compiler_params=pltpu.CompilerParams(use_tc_tiling_on_sc=False),
        out_type=jax.ShapeDtypeStruct((NC, N_PAD, d), jnp.float32),
        scratch_types=[
            pltpu.VMEM((NBUF, CHUNK), jnp.int32),
            pltpu.VMEM((NBUF, CHUNK), jnp.int32),
            pltpu.VMEM((NBUF * CHUNK, d), jnp.float32),
            pltpu.VMEM_SHARED((N_PAD, d), jnp.float32),
            pltpu.SemaphoreType.DMA,
            pltpu.SemaphoreType.DMA,
            pltpu.SemaphoreType.DMA,
            pltpu.SemaphoreType.DMA,
            pltpu.SemaphoreType.DMA,
            pltpu.SemaphoreType.DMA,
        ],
    )
    def _segsum(table, src, dst, out, sidx, didx, rows, acc,
                si0, si1, di0, di1, r0, r1):
        sem_si = [si0, si1]
        sem_di = [di0, di1]
        sem_r = [r0, r1]
        cid = lax.axis_index("c")
        sid = lax.axis_index("s")
        wid = sid * NC + cid
        crow = wid * NCHUNK

        zv = jnp.zeros((16,), jnp.float32)

        def _zrow(i, carry):
            for c in range(d // 16):
                rows[i, pl.ds(c * 16, 16)] = zv
            return carry

        lax.fori_loop(0, NBUF * CHUNK, _zrow, 0)
        base = sid * ROWS_PER_TILE
        pltpu.sync_copy(rows.at[pl.ds(0, 256)], acc.at[pl.ds(base, 256)])
        pltpu.sync_copy(rows.at[pl.ds(0, 256)], acc.at[pl.ds(base + 256, 256)])
        pltpu.sync_copy(rows.at[pl.ds(0, 120)], acc.at[pl.ds(base + 512, 120)])
        plsc.subcore_barrier()

        for b in range(NBUF):
            pltpu.async_copy(src.at[crow + b], sidx.at[b], sem_si[b])
            pltpu.async_copy(dst.at[crow + b], didx.at[b], sem_di[b])
        pltpu.make_async_copy(src.at[crow], sidx.at[0], sem_si[0]).wait()
        pltpu.async_copy(table.at[sidx.at[0]], rows.at[pl.ds(0, CHUNK)], sem_r[0])

        def _pair(g, carry):
            for b in range(NBUF):
                j = g * NBUF + b
                nb = (b + 1) % NBUF
                jn = j + 1
                rv = rows.at[pl.ds(b * CHUNK, CHUNK)]
                rvn = rows.at[pl.ds(nb * CHUNK, CHUNK)]

                @pl.when(jn < NCHUNK)
                def _():
                    pltpu.make_async_copy(src.at[crow + jn], sidx.at[nb],
                                          sem_si[nb]).wait()
                    pltpu.async_copy(table.at[sidx.at[nb]], rvn, sem_r[nb])

                pltpu.make_async_copy(table.at[sidx.at[b]], rv, sem_r[b]).wait()
                pltpu.make_async_copy(dst.at[crow + j], didx.at[b],
                                      sem_di[b]).wait()
                pltpu.sync_copy(rv, acc.at[didx.at[b]], add=True)
                jf = j + NBUF

                @pl.when(jf < NCHUNK)
                def _():
                    pltpu.async_copy(src.at[crow + jf], sidx.at[b], sem_si[b])
                    pltpu.async_copy(dst.at[crow + jf], didx.at[b], sem_di[b])

            return carry

        lax.fori_loop(0, NCHUNK // NBUF, _pair, 0)
        plsc.subcore_barrier()

        pltpu.sync_copy(acc.at[pl.ds(base, ROWS_PER_TILE)],
                        out.at[cid, pl.ds(base, ROWS_PER_TILE)])

    return _segsum


_segsum128 = _make_segsum(D_IN)
_segsum32 = _make_segsum(DIM)


def _bn(h, g, be):
    mu = jnp.mean(h, axis=0, keepdims=True)
    var = jnp.mean((h - mu) ** 2, axis=0, keepdims=True)
    return (h - mu) / jnp.sqrt(var + BN_EPS) * g + be


def _dense1_body(x_ref, p_ref, w1a_ref, b1a_ref, w1b_ref, b1b_ref, g1_ref,
                 be1_ref, o_ref):
    p = p_ref[...]
    h = x_ref[...] + p[0, :N_NODES] + p[1, :N_NODES]
    h = jnp.maximum(
        jnp.dot(h, w1a_ref[...], preferred_element_type=jnp.float32)
        + b1a_ref[...], 0.0)
    h = jnp.dot(h, w1b_ref[...], preferred_element_type=jnp.float32) + b1b_ref[...]
    h = jnp.maximum(h, 0.0)
    o_ref[...] = _bn(h, g1_ref[...], be1_ref[...])


_dense1 = pl.pallas_call(
    _dense1_body,
    out_shape=jax.ShapeDtypeStruct((N_NODES, DIM), jnp.float32),
)


def _dense2_body(h_ref, p_ref, w2a_ref, b2a_ref, w2b_ref, b2b_ref, g2_ref,
                 be2_ref, o_ref):
    p = p_ref[...]
    z = h_ref[...] + p[0, :N_NODES] + p[1, :N_NODES]
    t = jnp.maximum(
        jnp.dot(z, w2a_ref[...], preferred_element_type=jnp.float32)
        + b2a_ref[...], 0.0)
    t = jnp.dot(t, w2b_ref[...], preferred_element_type=jnp.float32) + b2b_ref[...]
    t = jnp.maximum(t, 0.0)
    o_ref[...] = _bn(t, g2_ref[...], be2_ref[...])


_dense2 = pl.pallas_call(
    _dense2_body,
    out_shape=jax.ShapeDtypeStruct((N_NODES, D_IN), jnp.float32),
)


def kernel(x, edge_index, W1a, b1a, W1b, b1b, g1, be1, W2a, b2a, W2b, b2b, g2, be2):
    src = edge_index[0].astype(jnp.int32).reshape(NW, E_PER_W)
    dst = edge_index[1].astype(jnp.int32).reshape(NW, E_PER_W)
    pad_src = jnp.zeros((NW, E_PAD_W - E_PER_W), jnp.int32)
    pad_dst = jnp.full((NW, E_PAD_W - E_PER_W), N_NODES, jnp.int32)
    src = jnp.concatenate([src, pad_src], axis=1).reshape(NW * NCHUNK, CHUNK)
    dst = jnp.concatenate([dst, pad_dst], axis=1).reshape(NW * NCHUNK, CHUNK)

    p1 = _segsum128(x, src, dst)
    h1 = _dense1(x, p1, W1a, b1a.reshape(1, DIM), W1b, b1b.reshape(1, DIM),
                 g1.reshape(1, DIM), be1.reshape(1, DIM))
    p2 = _segsum32(h1, src, dst)
    out = _dense2(h1, p2, W2a, b2a.reshape(1, DIM), W2b,
                  b2b.reshape(1, D_IN), g2.reshape(1, D_IN),
                  be2.reshape(1, D_IN))
    return out


# speedup vs baseline: 7.9083x; 1.4214x over previous
import functools

import jax
import jax.numpy as jnp
from jax import lax
from jax.experimental import pallas as pl
from jax.experimental.pallas import tpu as pltpu
from jax.experimental.pallas import tpu_sc as plsc

N_NODES = 10000
N_PAD = 10112
D_IN = 128
DIM = 32
HALF = D_IN // 2
BN_EPS = 1e-5
N_EDGES = 320000

NC = 2
NS = 16
NW = NC * NS
E_PER_W = N_EDGES // NW
CHUNK = 128
NCHUNK = 80
E_PAD_W = NCHUNK * CHUNK
NBUF = 8
GAHEAD = 4
ROWS_PER_TILE = N_PAD // NS

_mesh = plsc.VectorSubcoreMesh(core_axis_name="c", subcore_axis_name="s")


def _make_segsum(d, nch, feature_split):
    @functools.partial(
        pl.kernel,
        mesh=_mesh,
        compiler_params=pltpu.CompilerParams(use_tc_tiling_on_sc=False),
        out_type=jax.ShapeDtypeStruct((NC, N_PAD, d), jnp.float32),
        scratch_types=[
            pltpu.VMEM((NBUF, CHUNK), jnp.int32),
            pltpu.VMEM((NBUF, CHUNK), jnp.int32),
            pltpu.VMEM((NBUF * CHUNK, d), jnp.float32),
            pltpu.VMEM_SHARED((N_PAD, d), jnp.float32),
            pltpu.SemaphoreType.DMA((NBUF,)),
            pltpu.SemaphoreType.DMA((NBUF,)),
            pltpu.SemaphoreType.DMA((NBUF,)),
        ],
    )
    def _segsum(t0, t1, src, dst, out, sidx, didx, rows, acc,
                sem_si, sem_di, sem_r):
        cid = lax.axis_index("c")
        sid = lax.axis_index("s")
        if feature_split:
            crow = sid * nch
        else:
            crow = (sid * NC + cid) * nch

        def _gather_issue(slot, j):
            rv = rows.at[pl.ds(slot * CHUNK, CHUNK)]
            if feature_split:
                @pl.when(cid == 0)
                def _():
                    pltpu.async_copy(t0.at[sidx.at[slot]], rv, sem_r.at[slot])

                @pl.when(cid == 1)
                def _():
                    pltpu.async_copy(t1.at[sidx.at[slot]], rv, sem_r.at[slot])
            else:
                pltpu.async_copy(t0.at[sidx.at[slot]], rv, sem_r.at[slot])

        def _gather_wait(slot):
            pltpu.make_async_copy(t0.at[sidx.at[slot]],
                                  rows.at[pl.ds(slot * CHUNK, CHUNK)],
                                  sem_r.at[slot]).wait()

        zv = jnp.zeros((16,), jnp.float32)

        def _zrow(i, carry):
            for c in range(d // 16):
                rows[i, pl.ds(c * 16, 16)] = zv
            return carry

        lax.fori_loop(0, ROWS_PER_TILE, _zrow, 0)
        base = sid * ROWS_PER_TILE
        pltpu.sync_copy(rows.at[pl.ds(0, ROWS_PER_TILE)],
                        acc.at[pl.ds(base, ROWS_PER_TILE)])
        plsc.subcore_barrier()

        for k in range(NBUF):
            pltpu.async_copy(src.at[crow + k], sidx.at[k], sem_si.at[k])
        for k in range(GAHEAD):
            pltpu.async_copy(dst.at[crow + k], didx.at[k], sem_di.at[k])
        for k in range(GAHEAD):
            pltpu.make_async_copy(src.at[crow + k], sidx.at[k],
                                  sem_si.at[k]).wait()
            _gather_issue(k, k)

        def _group(g, carry):
            for b in range(NBUF):
                j = g * NBUF + b
                jg = j + GAHEAD
                bg = jg % NBUF

                @pl.when(jg < nch)
                def _():
                    pltpu.async_copy(dst.at[crow + jg], didx.at[bg],
                                     sem_di.at[bg])
                    pltpu.make_async_copy(src.at[crow + jg], sidx.at[bg],
                                          sem_si.at[bg]).wait()
                    _gather_issue(bg, jg)

                _gather_wait(b)
                jf = j + NBUF

                @pl.when(jf < nch)
                def _():
                    pltpu.async_copy(src.at[crow + jf], sidx.at[b],
                                     sem_si.at[b])

                pltpu.make_async_copy(dst.at[crow + j], didx.at[b],
                                      sem_di.at[b]).wait()
                pltpu.sync_copy(rows.at[pl.ds(b * CHUNK, CHUNK)],
                                acc.at[didx.at[b]], add=True)

            return carry

        lax.fori_loop(0, nch // NBUF, _group, 0)
        plsc.subcore_barrier()

        pltpu.sync_copy(acc.at[pl.ds(base, ROWS_PER_TILE)],
                        out.at[cid, pl.ds(base, ROWS_PER_TILE)])

    return _segsum


_segsum_l1 = _make_segsum(HALF, NCHUNK * 2, True)
_segsum_l2 = _make_segsum(DIM, NCHUNK, False)


def _presplit_body(x_ref, a_ref, b_ref):
    x = x_ref[...]
    a_ref[...] = x[:, :HALF]
    b_ref[...] = x[:, HALF:]


_presplit = pl.pallas_call(
    _presplit_body,
    out_shape=[jax.ShapeDtypeStruct((N_NODES, HALF), jnp.float32),
               jax.ShapeDtypeStruct((N_NODES, HALF), jnp.float32)],
)


def _bn(h, g, be):
    mu = jnp.mean(h, axis=0, keepdims=True)
    var = jnp.mean((h - mu) ** 2, axis=0, keepdims=True)
    return (h - mu) / jnp.sqrt(var + BN_EPS) * g + be


def _dense1_body(x_ref, p_ref, w1a_ref, b1a_ref, w1b_ref, b1b_ref, g1_ref,
                 be1_ref, o_ref):
    p = p_ref[...]
    agg = jnp.concatenate([p[0, :N_NODES], p[1, :N_NODES]], axis=1)
    h = x_ref[...] + agg
    h = jnp.maximum(
        jnp.dot(h, w1a_ref[...], preferred_element_type=jnp.float32)
        + b1a_ref[...], 0.0)
    h = jnp.dot(h, w1b_ref[...], preferred_element_type=jnp.float32) + b1b_ref[...]
    h = jnp.maximum(h, 0.0)
    o_ref[...] = _bn(h, g1_ref[...], be1_ref[...])


_dense1 = pl.pallas_call(
    _dense1_body,
    out_shape=jax.ShapeDtypeStruct((N_NODES, DIM), jnp.float32),
)


def _dense2_body(h_ref, p_ref, w2a_ref, b2a_ref, w2b_ref, b2b_ref, g2_ref,
                 be2_ref, o_ref):
    p = p_ref[...]
    z = h_ref[...] + p[0, :N_NODES] + p[1, :N_NODES]
    t = jnp.maximum(
        jnp.dot(z, w2a_ref[...], preferred_element_type=jnp.float32)
        + b2a_ref[...], 0.0)
    t = jnp.dot(t, w2b_ref[...], preferred_element_type=jnp.float32) + b2b_ref[...]
    t = jnp.maximum(t, 0.0)
    o_ref[...] = _bn(t, g2_ref[...], be2_ref[...])


_dense2 = pl.pallas_call(
    _dense2_body,
    out_shape=jax.ShapeDtypeStruct((N_NODES, D_IN), jnp.float32),
)


def kernel(x, edge_index, W1a, b1a, W1b, b1b, g1, be1, W2a, b2a, W2b, b2b, g2, be2):
    src = edge_index[0].astype(jnp.int32).reshape(NW, E_PER_W)
    dst = edge_index[1].astype(jnp.int32).reshape(NW, E_PER_W)
    pad_src = jnp.zeros((NW, E_PAD_W - E_PER_W), jnp.int32)
    pad_dst = jnp.full((NW, E_PAD_W - E_PER_W), N_NODES, jnp.int32)
    src = jnp.concatenate([src, pad_src], axis=1).reshape(NW * NCHUNK, CHUNK)
    dst = jnp.concatenate([dst, pad_dst], axis=1).reshape(NW * NCHUNK, CHUNK)

    xa, xb = _presplit(x)
    p1 = _segsum_l1(xa, xb, src, dst)
    h1 = _dense1(x, p1, W1a, b1a.reshape(1, DIM), W1b, b1b.reshape(1, DIM),
                 g1.reshape(1, DIM), be1.reshape(1, DIM))
    p2 = _segsum_l2(h1, h1, src, dst)
    out = _dense2(h1, p2, W2a, b2a.reshape(1, DIM), W2b,
                  b2b.reshape(1, D_IN), g2.reshape(1, D_IN),
                  be2.reshape(1, D_IN))
    return out
